# baseline (speedup 1.0000x reference)
import functools

import jax
import jax.numpy as jnp
from jax import lax
from jax.experimental import pallas as pl
from jax.experimental.pallas import tpu as pltpu
from jax.experimental.pallas import tpu_sc as plsc

_NC = 2
_NS = 16
_NW = _NC * _NS

_CH = 512
_SUB = 128
_K = _CH // _SUB


@functools.partial(jax.jit, static_argnums=(2, 3, 4, 5))
def _gather(table, flat_idx, V, D, B, NB):
    F = B // NB
    b_per_w = B // _NW
    n_ch = b_per_w // _CH
    assert n_ch >= 2 and n_ch % 2 == 0
    assert NB % _CH == 0 and b_per_w % _CH == 0
    mesh = plsc.VectorSubcoreMesh(core_axis_name="c", subcore_axis_name="s")

    DT = D // 8
    BT = NB // 128
    CT = _CH // 128

    @functools.partial(
        pl.kernel,
        out_type=jax.ShapeDtypeStruct((F, DT, BT, 8, 128), jnp.float32),
        mesh=mesh,
        scratch_types=[
            pltpu.VMEM((_CH,), jnp.int32),
            pltpu.VMEM((_CH,), jnp.int32),
            pltpu.VMEM((_CH, D), jnp.float32),
            pltpu.VMEM((_CH, D), jnp.float32),
            pltpu.VMEM((DT, CT, 8, 128), jnp.float32),
            pltpu.VMEM((DT, CT, 8, 128), jnp.float32),
            pltpu.SemaphoreType.DMA,
            pltpu.SemaphoreType.DMA,
            pltpu.SemaphoreType.DMA,
            pltpu.SemaphoreType.DMA,
        ],
        compiler_params=pltpu.CompilerParams(use_tc_tiling_on_sc=False,
                                             needs_layout_passes=False),
    )
    def k(table_hbm, idx_hbm, out_hbm, idx0, idx1, rows0, rows1, rt0, rt1,
          gsem0, gsem1, osem0, osem1):
        wid = lax.axis_index("s") * _NC + lax.axis_index("c")
        base = wid * b_per_w
        idx = (idx0, idx1)
        rows = (rows0, rows1)
        rt = (rt0, rt1)
        gsem = (gsem0, gsem1)
        osem = (osem0, osem1)

        def load_idx(p, g):
            pltpu.sync_copy(idx_hbm.at[pl.ds(base + g * _CH, _CH)], idx[p])

        def fire(p):
            for j in range(_K):
                pltpu.async_copy(
                    table_hbm.at[idx[p].at[pl.ds(j * _SUB, _SUB)]],
                    rows[p].at[pl.ds(j * _SUB, _SUB), :],
                    gsem[p],
                )

        def drain(p):
            for j in range(_K):
                pltpu.make_async_copy(
                    table_hbm.at[idx[p].at[pl.ds(j * _SUB, _SUB)]],
                    rows[p].at[pl.ds(j * _SUB, _SUB), :],
                    gsem[p],
                ).wait()

        def transpose(p):
            iota = lax.iota(jnp.int32, 16)
            dvecs = [(d0 + iota) % D for d0 in range(D)]

            def tbody(h, carry):
                for u in range(2):
                    m = 2 * h + u
                    b_ids = m * 16 + iota
                    jjv = jnp.full((16,), m >> 3, jnp.int32)
                    cv = (m & 7) * 16 + iota
                    for d0 in range(D):
                        dv = dvecs[d0]
                        v = plsc.load_gather(rows[p], [b_ids, dv])
                        plsc.store_scatter(
                            rt[p], [dv >> 3, jjv, dv & 7, cv], v)
                return carry

            lax.fori_loop(0, _CH // 32, tbody, 0)

        def store(p, g):
            j0 = base + g * _CH
            f = j0 // NB
            jt0 = (j0 % NB) // 128
            for t in range(DT):
                pltpu.async_copy(
                    rt[p].at[t],
                    out_hbm.at[f, t, pl.ds(jt0, CT)],
                    osem[p],
                )

        def wait_store(p, g):
            j0 = base + g * _CH
            f = j0 // NB
            jt0 = (j0 % NB) // 128
            for t in range(DT):
                pltpu.make_async_copy(
                    rt[p].at[t],
                    out_hbm.at[f, t, pl.ds(jt0, CT)],
                    osem[p],
                ).wait()

        load_idx(0, 0)
        fire(0)
        load_idx(1, 1)
        fire(1)
        drain(0)
        transpose(0)
        store(0, 0)

        def body(i, carry):
            g = 2 * i
            load_idx(0, g)
            fire(0)
            drain(1)
            transpose(1)
            wait_store(0, g - 2)
            store(1, g - 1)
            load_idx(1, g + 1)
            fire(1)
            drain(0)
            transpose(0)
            wait_store(1, g - 1)
            store(0, g)
            return carry

        lax.fori_loop(1, n_ch // 2, body, 0)

        drain(1)
        transpose(1)
        wait_store(0, n_ch - 2)
        store(1, n_ch - 1)
        wait_store(1, n_ch - 1)

    return k(table, flat_idx)


def kernel(indices, table):
    Bc, F = indices.shape
    V, D = table.shape
    B = Bc * F
    flat = indices.T.reshape(B)
    out5 = _gather(table, flat, V, D, B, Bc)
    return out5.transpose(2, 4, 0, 1, 3).reshape(Bc, F, D)


# speedup vs baseline: 1.1146x; 1.1146x over previous
import functools

import jax
import jax.numpy as jnp
from jax import lax
from jax.experimental import pallas as pl
from jax.experimental.pallas import tpu as pltpu
from jax.experimental.pallas import tpu_sc as plsc

_NC = 2
_NS = 16
_NW = _NC * _NS

_CH = 512
_SUB = 128
_K = _CH // _SUB


@functools.partial(jax.jit, static_argnums=(2, 3, 4, 5))
def _gather(table, flat_idx, V, D, B, NB):
    F = B // NB
    b_per_w = B // _NW
    n_ch = b_per_w // _CH
    assert n_ch >= 2 and n_ch % 2 == 0
    assert NB % _CH == 0 and b_per_w % _CH == 0
    mesh = plsc.VectorSubcoreMesh(core_axis_name="c", subcore_axis_name="s")

    DT = D // 8
    BT = NB // 128
    CT = _CH // 128

    @functools.partial(
        pl.kernel,
        out_type=jax.ShapeDtypeStruct((F, DT, BT, 8, 128), jnp.float32),
        mesh=mesh,
        scratch_types=[
            pltpu.VMEM((_CH,), jnp.int32),
            pltpu.VMEM((_CH,), jnp.int32),
            pltpu.VMEM((_CH, D), jnp.float32),
            pltpu.VMEM((_CH, D), jnp.float32),
            pltpu.VMEM((DT, CT, 8, 128), jnp.float32),
            pltpu.VMEM((DT, CT, 8, 128), jnp.float32),
            pltpu.SemaphoreType.DMA,
            pltpu.SemaphoreType.DMA,
            pltpu.SemaphoreType.DMA,
            pltpu.SemaphoreType.DMA,
        ],
        compiler_params=pltpu.CompilerParams(use_tc_tiling_on_sc=False,
                                             needs_layout_passes=False),
    )
    def k(table_hbm, idx_hbm, out_hbm, idx0, idx1, rows0, rows1, rt0, rt1,
          gsem0, gsem1, osem0, osem1):
        wid = lax.axis_index("s") * _NC + lax.axis_index("c")
        base = wid * b_per_w
        idx = (idx0, idx1)
        rows = (rows0, rows1)
        rt = (rt0, rt1)
        gsem = (gsem0, gsem1)
        osem = (osem0, osem1)

        def load_idx(p, g):
            pltpu.sync_copy(idx_hbm.at[pl.ds(base + g * _CH, _CH)], idx[p])

        def fire(p):
            for j in range(_K):
                pltpu.async_copy(
                    table_hbm.at[idx[p].at[pl.ds(j * _SUB, _SUB)]],
                    rows[p].at[pl.ds(j * _SUB, _SUB), :],
                    gsem[p],
                )

        def drain(p):
            for j in range(_K):
                pltpu.make_async_copy(
                    table_hbm.at[idx[p].at[pl.ds(j * _SUB, _SUB)]],
                    rows[p].at[pl.ds(j * _SUB, _SUB), :],
                    gsem[p],
                ).wait()

        def transpose(p):
            iota = lax.iota(jnp.int32, 16)
            dvecs = [(d0 + iota) % D for d0 in range(D)]

            def tbody(m, carry):
                b_ids = m * 16 + iota
                jjv = jnp.full((16,), m >> 3, jnp.int32)
                cv = (m & 7) * 16 + iota
                for d0 in range(D):
                    dv = dvecs[d0]
                    v = plsc.load_gather(rows[p], [b_ids, dv])
                    plsc.store_scatter(
                        rt[p], [dv >> 3, jjv, dv & 7, cv], v)
                return carry

            lax.fori_loop(0, _CH // 16, tbody, 0)

        def store(p, g):
            j0 = base + g * _CH
            f = j0 // NB
            jt0 = (j0 % NB) // 128
            for t in range(DT):
                pltpu.async_copy(
                    rt[p].at[t],
                    out_hbm.at[f, t, pl.ds(jt0, CT)],
                    osem[p],
                )

        def wait_store(p, g):
            j0 = base + g * _CH
            f = j0 // NB
            jt0 = (j0 % NB) // 128
            for t in range(DT):
                pltpu.make_async_copy(
                    rt[p].at[t],
                    out_hbm.at[f, t, pl.ds(jt0, CT)],
                    osem[p],
                ).wait()

        load_idx(0, 0)
        fire(0)
        load_idx(1, 1)
        fire(1)
        drain(0)
        transpose(0)
        store(0, 0)

        def body(i, carry):
            g = 2 * i
            load_idx(0, g)
            fire(0)
            drain(1)
            transpose(1)
            wait_store(0, g - 2)
            store(1, g - 1)
            load_idx(1, g + 1)
            fire(1)
            drain(0)
            transpose(0)
            wait_store(1, g - 1)
            store(0, g)
            return carry

        lax.fori_loop(1, n_ch // 2, body, 0)

        drain(1)
        transpose(1)
        wait_store(0, n_ch - 2)
        store(1, n_ch - 1)
        wait_store(1, n_ch - 1)

    return k(table, flat_idx)


def kernel(indices, table):
    Bc, F = indices.shape
    V, D = table.shape
    B = Bc * F
    flat = indices.T.reshape(B)
    out5 = _gather(table, flat, V, D, B, Bc)
    return out5.transpose(2, 4, 0, 1, 3).reshape(Bc, F, D)
